# baseline (speedup 1.0000x reference)
import functools

import jax
import jax.numpy as jnp
from jax import lax
from jax.experimental import pallas as pl
from jax.experimental.pallas import tpu as pltpu
from jax.experimental.pallas import tpu_sc as plsc

_VOCAB = 100000
_D = 300
_DP = 304
_B = 16384
_W = 3
_NC, _NS = 2, 16
_NW = _NC * _NS
_ROWS = _B * _W
_PER_W = _ROWS // _NW
_CHUNK = 128
_NCHUNK = _PER_W // _CHUNK


@functools.partial(
    pl.kernel,
    out_type=jax.ShapeDtypeStruct((_ROWS, _DP), jnp.float32),
    mesh=plsc.VectorSubcoreMesh(core_axis_name="c", subcore_axis_name="s"),
    compiler_params=pltpu.CompilerParams(use_tc_tiling_on_sc=False),
    scratch_types=[
        pltpu.VMEM((_NCHUNK, _CHUNK), jnp.int32),
        pltpu.VMEM((2, _CHUNK, _DP), jnp.float32),
        pltpu.SemaphoreType.DMA,
        pltpu.SemaphoreType.DMA,
    ],
)
def _gather_rows(idx_hbm, table_hbm, out_hbm, idx_v, rows_v, sem0, sem1):
    wid = lax.axis_index("s") * _NC + lax.axis_index("c")
    base = wid * _PER_W
    pltpu.sync_copy(idx_hbm.at[wid], idx_v)
    sems = (sem0, sem1)
    copies = [None, None]
    copies[0] = pltpu.async_copy(table_hbm.at[idx_v.at[0]], rows_v.at[0], sems[0])
    for j in range(_NCHUNK):
        b = j % 2
        if j + 1 < _NCHUNK:
            nb = (j + 1) % 2
            copies[nb] = pltpu.async_copy(
                table_hbm.at[idx_v.at[j + 1]], rows_v.at[nb], sems[nb]
            )
        copies[b].wait()
        pltpu.sync_copy(rows_v.at[b], out_hbm.at[pl.ds(base + j * _CHUNK, _CHUNK)])


def kernel(class_labels, captions, table):
    table_p = jnp.pad(table, ((0, 0), (0, _DP - _D)))
    idx = captions.reshape(_NW, _NCHUNK, _CHUNK)
    out = _gather_rows(idx, table_p)
    return out[:, :_D].reshape(_B, _W * _D)


# speedup vs baseline: 1.3108x; 1.3108x over previous
import functools

import jax
import jax.numpy as jnp
from jax import lax
from jax.experimental import pallas as pl
from jax.experimental.pallas import tpu as pltpu
from jax.experimental.pallas import tpu_sc as plsc

_VOCAB = 100000
_D = 300
_DP = 384
_B = 16384
_W = 3
_NC, _NS = 2, 16
_NW = _NC * _NS
_ROWS = _B * _W
_PER_W = _ROWS // _NW
_CHUNK = 128
_NCHUNK = _PER_W // _CHUNK


@functools.partial(
    pl.kernel,
    out_type=jax.ShapeDtypeStruct((_ROWS, _DP), jnp.float32),
    mesh=plsc.VectorSubcoreMesh(core_axis_name="c", subcore_axis_name="s"),
    scratch_types=[
        pltpu.VMEM((_PER_W,), jnp.int32),
        pltpu.VMEM((2, _CHUNK, _DP), jnp.float32),
        pltpu.SemaphoreType.DMA,
        pltpu.SemaphoreType.DMA,
    ],
)
def _gather_rows(idx_hbm, table_hbm, out_hbm, idx_v, rows_v, sem0, sem1):
    wid = lax.axis_index("s") * _NC + lax.axis_index("c")
    base = wid * _PER_W
    pltpu.sync_copy(idx_hbm.at[pl.ds(base, _PER_W)], idx_v)
    sems = (sem0, sem1)
    copies = [None, None]
    copies[0] = pltpu.async_copy(
        table_hbm.at[idx_v.at[pl.ds(0, _CHUNK)]], rows_v.at[0], sems[0]
    )
    for j in range(_NCHUNK):
        b = j % 2
        if j + 1 < _NCHUNK:
            nb = (j + 1) % 2
            copies[nb] = pltpu.async_copy(
                table_hbm.at[idx_v.at[pl.ds((j + 1) * _CHUNK, _CHUNK)]],
                rows_v.at[nb],
                sems[nb],
            )
        copies[b].wait()
        pltpu.sync_copy(rows_v.at[b], out_hbm.at[pl.ds(base + j * _CHUNK, _CHUNK)])


def kernel(class_labels, captions, table):
    table_p = jnp.pad(table, ((0, 0), (0, _DP - _D)))
    idx = captions.reshape(_ROWS)
    out = _gather_rows(idx, table_p)
    return out[:, :_D].reshape(_B, _W * _D)


# speedup vs baseline: 3.1042x; 2.3682x over previous
import functools

import jax
import jax.numpy as jnp
from jax import lax
from jax.experimental import pallas as pl
from jax.experimental.pallas import tpu as pltpu
from jax.experimental.pallas import tpu_sc as plsc

_VOCAB = 100000
_D = 300
_DP = 384
_B = 16384
_W = 3
_NC, _NS = 2, 16
_NW = _NC * _NS
_ROWS = _B * _W
_PER_W = _ROWS // _NW
_CHUNK = 128
_NCHUNK = _PER_W // _CHUNK


@functools.partial(
    pl.kernel,
    out_type=jax.ShapeDtypeStruct((_ROWS, _DP), jnp.float32),
    mesh=plsc.VectorSubcoreMesh(core_axis_name="c", subcore_axis_name="s"),
    scratch_types=[
        pltpu.VMEM((_PER_W,), jnp.int32),
        pltpu.VMEM((2, _CHUNK, _DP), jnp.float32),
        pltpu.SemaphoreType.DMA,
        pltpu.SemaphoreType.DMA,
    ],
)
def _gather_rows(idx_hbm, table_hbm, out_hbm, idx_v, rows_v, sem0, sem1):
    wid = lax.axis_index("s") * _NC + lax.axis_index("c")
    base = wid * _PER_W
    pltpu.sync_copy(idx_hbm.at[pl.ds(base, _PER_W)], idx_v)
    sems = (sem0, sem1)
    copies = [None, None]
    copies[0] = pltpu.async_copy(
        table_hbm.at[idx_v.at[pl.ds(0, _CHUNK)]], rows_v.at[0], sems[0]
    )
    for j in range(_NCHUNK):
        b = j % 2
        if j + 1 < _NCHUNK:
            nb = (j + 1) % 2
            copies[nb] = pltpu.async_copy(
                table_hbm.at[idx_v.at[pl.ds((j + 1) * _CHUNK, _CHUNK)]],
                rows_v.at[nb],
                sems[nb],
            )
        copies[b].wait()
        pltpu.sync_copy(rows_v.at[b], out_hbm.at[pl.ds(base + j * _CHUNK, _CHUNK)])


_TBLK = 1024


def _transpose_pad_body(tt_ref, out_ref):
    out_ref[:, :_D] = tt_ref[...].T
    out_ref[:, _D:] = jnp.zeros((_TBLK, _DP - _D), jnp.float32)


_transpose_pad = pl.pallas_call(
    _transpose_pad_body,
    grid=(pl.cdiv(_VOCAB, _TBLK),),
    in_specs=[pl.BlockSpec((_D, _TBLK), lambda g: (0, g))],
    out_specs=pl.BlockSpec((_TBLK, _DP), lambda g: (g, 0)),
    out_shape=jax.ShapeDtypeStruct((_VOCAB, _DP), jnp.float32),
)


def kernel(class_labels, captions, table):
    table_p = _transpose_pad(jnp.swapaxes(table, 0, 1))
    idx = captions.reshape(_ROWS)
    out = _gather_rows(idx, table_p)
    return out[:, :_D].reshape(_B, _W * _D)


# speedup vs baseline: 3.4044x; 1.0967x over previous
import functools

import jax
import jax.numpy as jnp
from jax import lax
from jax.experimental import pallas as pl
from jax.experimental.pallas import tpu as pltpu
from jax.experimental.pallas import tpu_sc as plsc

_VOCAB = 100000
_D = 300
_DP = 384
_B = 16384
_W = 3
_NC, _NS = 2, 16
_NW = _NC * _NS
_ROWS = _B * _W
_PER_W = _ROWS // _NW
_CHUNK = 128
_NCHUNK = _PER_W // _CHUNK


@functools.partial(
    pl.kernel,
    out_type=jax.ShapeDtypeStruct((_ROWS, _DP), jnp.float32),
    mesh=plsc.VectorSubcoreMesh(core_axis_name="c", subcore_axis_name="s"),
    scratch_types=[
        pltpu.VMEM((_PER_W,), jnp.int32),
        pltpu.VMEM((2, _CHUNK, _DP), jnp.float32),
        pltpu.SemaphoreType.DMA,
        pltpu.SemaphoreType.DMA,
    ],
)
def _gather_rows(idx_hbm, table_hbm, out_hbm, idx_v, rows_v, sem0, sem1):
    wid = lax.axis_index("s") * _NC + lax.axis_index("c")
    base = wid * _PER_W
    pltpu.sync_copy(idx_hbm.at[pl.ds(base, _PER_W)], idx_v)
    sems = (sem0, sem1)
    copies = [None, None]
    copies[0] = pltpu.async_copy(
        table_hbm.at[idx_v.at[pl.ds(0, _CHUNK)]], rows_v.at[0], sems[0]
    )
    for j in range(_NCHUNK):
        b = j % 2
        if j + 1 < _NCHUNK:
            nb = (j + 1) % 2
            copies[nb] = pltpu.async_copy(
                table_hbm.at[idx_v.at[pl.ds((j + 1) * _CHUNK, _CHUNK)]],
                rows_v.at[nb],
                sems[nb],
            )
        copies[b].wait()
        pltpu.sync_copy(rows_v.at[b], out_hbm.at[pl.ds(base + j * _CHUNK, _CHUNK)])


_TBLK = 2048


def _transpose_pad_body(tt_ref, out_ref):
    out_ref[:, :_D] = tt_ref[...].T
    out_ref[:, _D:] = jnp.zeros((_TBLK, _DP - _D), jnp.float32)


_transpose_pad = pl.pallas_call(
    _transpose_pad_body,
    grid=(pl.cdiv(_VOCAB, _TBLK),),
    in_specs=[pl.BlockSpec((_D, _TBLK), lambda g: (0, g))],
    out_specs=pl.BlockSpec((_TBLK, _DP), lambda g: (g, 0)),
    out_shape=jax.ShapeDtypeStruct((_VOCAB, _DP), jnp.float32),
)


def kernel(class_labels, captions, table):
    table_p = _transpose_pad(jnp.swapaxes(table, 0, 1))
    idx = captions.reshape(_ROWS)
    out = _gather_rows(idx, table_p)
    return out[:, :_D].reshape(_B, _W * _D)


# speedup vs baseline: 4.6929x; 1.3785x over previous
import functools

import jax
import jax.numpy as jnp
from jax import lax
from jax.experimental import pallas as pl
from jax.experimental.pallas import tpu as pltpu
from jax.experimental.pallas import tpu_sc as plsc

_VOCAB = 100000
_D = 300
_DP = 384
_B = 16384
_W = 3
_NC, _NS = 2, 16
_NW = _NC * _NS
_ROWS = _B * _W
_PER_W = _ROWS // _NW
_CHUNK = 128
_NCHUNK = _PER_W // _CHUNK


@functools.partial(
    pl.kernel,
    out_type=jax.ShapeDtypeStruct((_ROWS, _DP), jnp.float32),
    mesh=plsc.VectorSubcoreMesh(core_axis_name="c", subcore_axis_name="s"),
    scratch_types=[
        pltpu.VMEM((_PER_W,), jnp.int32),
        pltpu.VMEM((2, _CHUNK, _DP), jnp.float32),
        pltpu.SemaphoreType.DMA,
        pltpu.SemaphoreType.DMA,
    ],
)
def _gather_rows(idx_hbm, table_hbm, out_hbm, idx_v, rows_v, sem0, sem1):
    wid = lax.axis_index("s") * _NC + lax.axis_index("c")
    base = wid * _PER_W
    pltpu.sync_copy(idx_hbm.at[pl.ds(base, _PER_W)], idx_v)
    sems = (sem0, sem1)
    copies = [None, None]
    copies[0] = pltpu.async_copy(
        table_hbm.at[idx_v.at[pl.ds(0, _CHUNK)]], rows_v.at[0], sems[0]
    )
    for j in range(_NCHUNK):
        b = j % 2
        if j + 1 < _NCHUNK:
            nb = (j + 1) % 2
            copies[nb] = pltpu.async_copy(
                table_hbm.at[idx_v.at[pl.ds((j + 1) * _CHUNK, _CHUNK)]],
                rows_v.at[nb],
                sems[nb],
            )
        copies[b].wait()
        pltpu.sync_copy(rows_v.at[b], out_hbm.at[pl.ds(base + j * _CHUNK, _CHUNK)])


_TBLK = 4096


def _transpose_pad_body(tt_ref, out_ref):
    out_ref[:, :_D] = tt_ref[...].T
    out_ref[:, _D:] = jnp.zeros((_TBLK, _DP - _D), jnp.float32)


_transpose_pad = pl.pallas_call(
    _transpose_pad_body,
    grid=(pl.cdiv(_VOCAB, _TBLK),),
    in_specs=[pl.BlockSpec((_D, _TBLK), lambda g: (0, g))],
    out_specs=pl.BlockSpec((_TBLK, _DP), lambda g: (g, 0)),
    out_shape=jax.ShapeDtypeStruct((_VOCAB, _DP), jnp.float32),
)


_RBLK = 512


def _repack_body(x0_ref, x1_ref, x2_ref, out_ref):
    y = jnp.concatenate(
        [x0_ref[:, :_D], x1_ref[:, :_D], x2_ref[:, :_D]], axis=1
    )
    out_ref[...] = y.T


_repack = pl.pallas_call(
    _repack_body,
    grid=(_B // _RBLK,),
    in_specs=[
        pl.BlockSpec((_RBLK, _DP), lambda g, w=w: (w * (_B // _RBLK) + g, 0))
        for w in range(_W)
    ],
    out_specs=pl.BlockSpec((_W * _D, _RBLK), lambda g: (0, g)),
    out_shape=jax.ShapeDtypeStruct((_W * _D, _B), jnp.float32),
)


def kernel(class_labels, captions, table):
    table_p = _transpose_pad(jnp.swapaxes(table, 0, 1))
    idx = jnp.swapaxes(captions, 0, 1).reshape(_ROWS)
    out = _gather_rows(idx, table_p)
    res_t = _repack(out, out, out)
    return jnp.swapaxes(res_t, 0, 1)


# speedup vs baseline: 4.7737x; 1.0172x over previous
import functools

import jax
import jax.numpy as jnp
from jax import lax
from jax.experimental import pallas as pl
from jax.experimental.pallas import tpu as pltpu
from jax.experimental.pallas import tpu_sc as plsc

_VOCAB = 100000
_D = 300
_DP = 384
_B = 16384
_HB = _B // 2
_W = 3
_NC, _NS = 2, 16
_NW = _NC * _NS
_HROWS = _HB * _W
_PER_W = _HROWS // _NW
_CHUNK = 128
_NCHUNK = _PER_W // _CHUNK


@functools.partial(
    pl.kernel,
    out_type=jax.ShapeDtypeStruct((_HROWS, _DP), jnp.float32),
    mesh=plsc.VectorSubcoreMesh(core_axis_name="c", subcore_axis_name="s"),
    scratch_types=[
        pltpu.VMEM((_PER_W,), jnp.int32),
        pltpu.VMEM((2, _CHUNK, _DP), jnp.float32),
        pltpu.SemaphoreType.DMA,
        pltpu.SemaphoreType.DMA,
    ],
)
def _gather_rows(idx_hbm, table_hbm, out_hbm, idx_v, rows_v, sem0, sem1):
    wid = lax.axis_index("s") * _NC + lax.axis_index("c")
    base = wid * _PER_W
    pltpu.sync_copy(idx_hbm.at[pl.ds(base, _PER_W)], idx_v)
    sems = (sem0, sem1)
    copies = [None, None]
    copies[0] = pltpu.async_copy(
        table_hbm.at[idx_v.at[pl.ds(0, _CHUNK)]], rows_v.at[0], sems[0]
    )
    for j in range(_NCHUNK):
        b = j % 2
        if j + 1 < _NCHUNK:
            nb = (j + 1) % 2
            copies[nb] = pltpu.async_copy(
                table_hbm.at[idx_v.at[pl.ds((j + 1) * _CHUNK, _CHUNK)]],
                rows_v.at[nb],
                sems[nb],
            )
        copies[b].wait()
        pltpu.sync_copy(rows_v.at[b], out_hbm.at[pl.ds(base + j * _CHUNK, _CHUNK)])


_TBLK = 4096


def _transpose_pad_body(tt_ref, out_ref):
    out_ref[:, :_D] = tt_ref[...].T
    out_ref[:, _D:] = jnp.zeros((_TBLK, _DP - _D), jnp.float32)


_transpose_pad = pl.pallas_call(
    _transpose_pad_body,
    grid=(pl.cdiv(_VOCAB, _TBLK),),
    in_specs=[pl.BlockSpec((_D, _TBLK), lambda g: (0, g))],
    out_specs=pl.BlockSpec((_TBLK, _DP), lambda g: (g, 0)),
    out_shape=jax.ShapeDtypeStruct((_VOCAB, _DP), jnp.float32),
)


_RBLK = 512


def _repack_half_body(x0_ref, x1_ref, x2_ref, out_ref):
    y = jnp.concatenate(
        [x0_ref[:, :_D], x1_ref[:, :_D], x2_ref[:, :_D]], axis=1
    )
    out_ref[...] = y.T


def _acc_repack_half_body(acc_ref, x0_ref, x1_ref, x2_ref, out_ref):
    del acc_ref
    _repack_half_body(x0_ref, x1_ref, x2_ref, out_ref)


def _make_repack(half):
    word_specs = [
        pl.BlockSpec((_RBLK, _DP), lambda g, w=w: (w * (_HB // _RBLK) + g, 0))
        for w in range(_W)
    ]
    out_spec = pl.BlockSpec(
        (_W * _D, _RBLK), lambda g, h=half: (0, h * (_HB // _RBLK) + g)
    )
    out_type = jax.ShapeDtypeStruct((_W * _D, _B), jnp.float32)
    if half == 0:
        return pl.pallas_call(
            _repack_half_body,
            grid=(_HB // _RBLK,),
            in_specs=word_specs,
            out_specs=out_spec,
            out_shape=out_type,
        )
    return pl.pallas_call(
        _acc_repack_half_body,
        grid=(_HB // _RBLK,),
        in_specs=[pl.BlockSpec(memory_space=pltpu.MemorySpace.HBM)] + word_specs,
        out_specs=out_spec,
        out_shape=out_type,
        input_output_aliases={0: 0},
    )


_repack0 = _make_repack(0)
_repack1 = _make_repack(1)


def kernel(class_labels, captions, table):
    table_p = _transpose_pad(jnp.swapaxes(table, 0, 1))
    idx_t = jnp.swapaxes(captions, 0, 1)
    idx0 = idx_t[:, :_HB].reshape(_HROWS)
    idx1 = idx_t[:, _HB:].reshape(_HROWS)
    g0 = _gather_rows(idx0, table_p)
    g1 = _gather_rows(idx1, table_p)
    acc = _repack0(g0, g0, g0)
    res_t = _repack1(acc, g1, g1, g1)
    return jnp.swapaxes(res_t, 0, 1)


# speedup vs baseline: 4.7934x; 1.0041x over previous
import functools

import jax
import jax.numpy as jnp
from jax import lax
from jax.experimental import pallas as pl
from jax.experimental.pallas import tpu as pltpu
from jax.experimental.pallas import tpu_sc as plsc

_VOCAB = 100000
_D = 300
_DP = 384
_B = 16384
_HB = _B // 2
_W = 3
_NC, _NS = 2, 16
_NW = _NC * _NS
_HROWS = _HB * _W
_PER_W = _HROWS // _NW
_CHUNK = 128
_NCHUNK = _PER_W // _CHUNK


@functools.partial(
    pl.kernel,
    out_type=jax.ShapeDtypeStruct((_HROWS, _DP), jnp.float32),
    mesh=plsc.VectorSubcoreMesh(core_axis_name="c", subcore_axis_name="s"),
    scratch_types=[
        pltpu.VMEM((_PER_W,), jnp.int32),
        pltpu.VMEM((2, _CHUNK, _DP), jnp.float32),
        pltpu.SemaphoreType.DMA,
        pltpu.SemaphoreType.DMA,
        pltpu.SemaphoreType.DMA,
        pltpu.SemaphoreType.DMA,
    ],
)
def _gather_rows(idx_hbm, table_hbm, out_hbm, idx_v, rows_v, sem0, sem1, wsem0, wsem1):
    wid = lax.axis_index("s") * _NC + lax.axis_index("c")
    base = wid * _PER_W
    pltpu.sync_copy(idx_hbm.at[pl.ds(base, _PER_W)], idx_v)
    sems = (sem0, sem1)
    wsems = (wsem0, wsem1)
    copies = [None, None]
    wcopies = [None, None]
    copies[0] = pltpu.async_copy(
        table_hbm.at[idx_v.at[pl.ds(0, _CHUNK)]], rows_v.at[0], sems[0]
    )
    for j in range(_NCHUNK):
        b = j % 2
        if j + 1 < _NCHUNK:
            nb = (j + 1) % 2
            if wcopies[nb] is not None:
                wcopies[nb].wait()
            copies[nb] = pltpu.async_copy(
                table_hbm.at[idx_v.at[pl.ds((j + 1) * _CHUNK, _CHUNK)]],
                rows_v.at[nb],
                sems[nb],
            )
        copies[b].wait()
        wcopies[b] = pltpu.async_copy(
            rows_v.at[b], out_hbm.at[pl.ds(base + j * _CHUNK, _CHUNK)], wsems[b]
        )
    for b in range(2):
        if wcopies[b] is not None:
            wcopies[b].wait()


_TBLK = 4096


def _transpose_pad_body(tt_ref, out_ref):
    out_ref[:, :_D] = tt_ref[...].T
    out_ref[:, _D:] = jnp.zeros((_TBLK, _DP - _D), jnp.float32)


_transpose_pad = pl.pallas_call(
    _transpose_pad_body,
    grid=(pl.cdiv(_VOCAB, _TBLK),),
    in_specs=[pl.BlockSpec((_D, _TBLK), lambda g: (0, g))],
    out_specs=pl.BlockSpec((_TBLK, _DP), lambda g: (g, 0)),
    out_shape=jax.ShapeDtypeStruct((_VOCAB, _DP), jnp.float32),
)


_RBLK = 512


def _repack_half_body(x0_ref, x1_ref, x2_ref, out_ref):
    y = jnp.concatenate(
        [x0_ref[:, :_D], x1_ref[:, :_D], x2_ref[:, :_D]], axis=1
    )
    out_ref[...] = y.T


def _acc_repack_half_body(acc_ref, x0_ref, x1_ref, x2_ref, out_ref):
    del acc_ref
    _repack_half_body(x0_ref, x1_ref, x2_ref, out_ref)


def _make_repack(half):
    word_specs = [
        pl.BlockSpec((_RBLK, _DP), lambda g, w=w: (w * (_HB // _RBLK) + g, 0))
        for w in range(_W)
    ]
    out_spec = pl.BlockSpec(
        (_W * _D, _RBLK), lambda g, h=half: (0, h * (_HB // _RBLK) + g)
    )
    out_type = jax.ShapeDtypeStruct((_W * _D, _B), jnp.float32)
    if half == 0:
        return pl.pallas_call(
            _repack_half_body,
            grid=(_HB // _RBLK,),
            in_specs=word_specs,
            out_specs=out_spec,
            out_shape=out_type,
        )
    return pl.pallas_call(
        _acc_repack_half_body,
        grid=(_HB // _RBLK,),
        in_specs=[pl.BlockSpec(memory_space=pltpu.MemorySpace.HBM)] + word_specs,
        out_specs=out_spec,
        out_shape=out_type,
        input_output_aliases={0: 0},
    )


_repack0 = _make_repack(0)
_repack1 = _make_repack(1)


def kernel(class_labels, captions, table):
    table_p = _transpose_pad(jnp.swapaxes(table, 0, 1))
    idx_t = jnp.swapaxes(captions, 0, 1)
    idx0 = idx_t[:, :_HB].reshape(_HROWS)
    idx1 = idx_t[:, _HB:].reshape(_HROWS)
    g0 = _gather_rows(idx0, table_p)
    g1 = _gather_rows(idx1, table_p)
    acc = _repack0(g0, g0, g0)
    res_t = _repack1(acc, g1, g1, g1)
    return jnp.swapaxes(res_t, 0, 1)
